# baseline (device time: 27448 ns/iter reference)
import jax
import jax.numpy as jnp
from jax import lax
from jax.experimental import pallas as pl
from jax.experimental.pallas import tpu as pltpu

N_DEV = 16


def kernel(x, w_mat, scale_x, scale_w):
    k_full, k_per = x.shape
    _, n = w_mat.shape
    m_per = k_full // N_DEV

    sx = scale_x.reshape(1, 1)
    sw = scale_w.reshape(1, 1)

    def body(x_ref, w_ref, sx_ref, sw_ref, out_ref, comm_ref,
             send_sems, recv_sems):
        my_i = lax.axis_index("i")

        barrier = pltpu.get_barrier_semaphore()
        for d in range(1, N_DEV):
            peer = lax.rem(my_i + d, N_DEV)
            pl.semaphore_signal(
                barrier, inc=1,
                device_id=(peer,), device_id_type=pl.DeviceIdType.MESH,
            )
        pl.semaphore_wait(barrier, N_DEV - 1)

        sends = []
        for d in range(1, N_DEV):
            j = lax.rem(my_i + d, N_DEV)
            rdma = pltpu.make_async_remote_copy(
                src_ref=x_ref.at[pl.ds(j * m_per, m_per), :],
                dst_ref=comm_ref.at[:, pl.ds(my_i * k_per, k_per)],
                send_sem=send_sems.at[d],
                recv_sem=recv_sems.at[d],
                device_id=(j,),
                device_id_type=pl.DeviceIdType.MESH,
            )
            rdma.start()
            sends.append(rdma)

        comm_ref[:, pl.ds(my_i * k_per, k_per)] = \
            x_ref[pl.ds(my_i * m_per, m_per), :]

        for d in range(1, N_DEV):
            src = lax.rem(my_i - d + N_DEV, N_DEV)
            recv = pltpu.make_async_remote_copy(
                src_ref=x_ref.at[pl.ds(src * m_per, m_per), :],
                dst_ref=comm_ref.at[:, pl.ds(src * k_per, k_per)],
                send_sem=send_sems.at[d],
                recv_sem=recv_sems.at[d],
                device_id=(src,),
                device_id_type=pl.DeviceIdType.MESH,
            )
            recv.wait_recv()

        acc = jnp.dot(comm_ref[:, :], w_ref[:, :],
                      preferred_element_type=jnp.int32)
        s = sx_ref[0, 0] * sw_ref[0, 0]
        y = acc.astype(jnp.float32) * s
        z = jnp.clip(y, -60.0, 60.0)
        out_ref[:, :] = y / (1.0 + jnp.exp(-z))

        for rdma in sends:
            rdma.wait_send()

    return pl.pallas_call(
        body,
        out_shape=jax.ShapeDtypeStruct((m_per, n), jnp.float32),
        in_specs=[
            pl.BlockSpec(memory_space=pltpu.VMEM),
            pl.BlockSpec(memory_space=pltpu.VMEM),
            pl.BlockSpec(memory_space=pltpu.VMEM),
            pl.BlockSpec(memory_space=pltpu.VMEM),
        ],
        out_specs=pl.BlockSpec(memory_space=pltpu.VMEM),
        scratch_shapes=[
            pltpu.VMEM((m_per, N_DEV * k_per), jnp.int8),
            pltpu.SemaphoreType.DMA((N_DEV,)),
            pltpu.SemaphoreType.DMA((N_DEV,)),
        ],
        compiler_params=pltpu.CompilerParams(collective_id=0),
    )(x, w_mat, sx, sw)


# device time: 26477 ns/iter; 1.0367x vs baseline; 1.0367x over previous
import jax
import jax.numpy as jnp
from jax import lax
from jax.experimental import pallas as pl
from jax.experimental.pallas import tpu as pltpu

N_DEV = 16


def kernel(x, w_mat, scale_x, scale_w):
    k_full, k_per = x.shape
    _, n = w_mat.shape
    m_per = k_full // N_DEV

    sx = scale_x.reshape(1, 1)
    sw = scale_w.reshape(1, 1)

    def body(x_ref, w_ref, sx_ref, sw_ref, out_ref, comm_ref, wbuf,
             send_sems, recv_sems, wsem, credit_sems):
        my_i = lax.axis_index("i")

        for d in range(1, N_DEV):
            peer = lax.rem(my_i + d, N_DEV)
            pl.semaphore_signal(
                credit_sems.at[d], inc=1,
                device_id=(peer,), device_id_type=pl.DeviceIdType.MESH,
            )

        barrier = pltpu.get_barrier_semaphore()
        pl.semaphore_signal(barrier, inc=1, device_id=(my_i,),
                            device_id_type=pl.DeviceIdType.MESH)
        pl.semaphore_wait(barrier, 1)

        w_copy = pltpu.make_async_copy(w_ref, wbuf, wsem)
        w_copy.start()

        sends = []
        for d in range(1, N_DEV):
            j = lax.rem(my_i + d, N_DEV)
            pl.semaphore_wait(credit_sems.at[d], 1)
            rdma = pltpu.make_async_remote_copy(
                src_ref=x_ref.at[pl.ds(j * m_per, m_per), :],
                dst_ref=comm_ref.at[:, pl.ds(my_i * k_per, k_per)],
                send_sem=send_sems.at[d],
                recv_sem=recv_sems.at[d],
                device_id=(j,),
                device_id_type=pl.DeviceIdType.MESH,
            )
            rdma.start()
            sends.append(rdma)

        comm_ref[:, pl.ds(my_i * k_per, k_per)] = \
            x_ref[pl.ds(my_i * m_per, m_per), :]

        for d in range(1, N_DEV):
            src = lax.rem(my_i - d + N_DEV, N_DEV)
            recv = pltpu.make_async_remote_copy(
                src_ref=x_ref.at[pl.ds(src * m_per, m_per), :],
                dst_ref=comm_ref.at[:, pl.ds(src * k_per, k_per)],
                send_sem=send_sems.at[d],
                recv_sem=recv_sems.at[d],
                device_id=(src,),
                device_id_type=pl.DeviceIdType.MESH,
            )
            recv.wait_recv()
        w_copy.wait()

        acc = jnp.dot(comm_ref[:, :], wbuf[:, :],
                      preferred_element_type=jnp.int32)
        s_ = sx_ref[0, 0] * sw_ref[0, 0]
        y = acc.astype(jnp.float32) * s_
        z = jnp.clip(y, -60.0, 60.0)
        out_ref[:, :] = y / (1.0 + jnp.exp(-z))

        for rdma in sends:
            rdma.wait_send()

    return pl.pallas_call(
        body,
        out_shape=jax.ShapeDtypeStruct((m_per, n), jnp.float32),
        in_specs=[
            pl.BlockSpec(memory_space=pltpu.VMEM),
            pl.BlockSpec(memory_space=pltpu.MemorySpace.HBM),
            pl.BlockSpec(memory_space=pltpu.VMEM),
            pl.BlockSpec(memory_space=pltpu.VMEM),
        ],
        out_specs=pl.BlockSpec(memory_space=pltpu.VMEM),
        scratch_shapes=[
            pltpu.VMEM((m_per, N_DEV * k_per), jnp.int8),
            pltpu.VMEM((k_full, n), jnp.int8),
            pltpu.SemaphoreType.DMA((N_DEV,)),
            pltpu.SemaphoreType.DMA((N_DEV,)),
            pltpu.SemaphoreType.DMA,
            pltpu.SemaphoreType.REGULAR((N_DEV,)),
        ],
        compiler_params=pltpu.CompilerParams(collective_id=0),
    )(x, w_mat, sx, sw)
